# baseline (speedup 1.0000x reference)
import jax
import jax.numpy as jnp
from jax.experimental import pallas as pl
from jax.experimental.pallas import tpu as pltpu

B, N, C, K = 2, 4096, 16, 16
R = 512
G = 8
W = N // G
NEG = -1e30


def _adj_kernel(xr_ref, xa_ref, out_ref):
    xr = xr_ref[0]
    xa = xa_ref[0]
    sq_a_half = 0.5 * jnp.sum(xa * xa, axis=-1, keepdims=True).T
    dots = jax.lax.dot_general(
        xr, xa, (((1,), (1,)), ((), ())),
        preferred_element_type=jnp.float32)
    s = sq_a_half - dots

    gmin = s[:, :W]
    for g in range(1, G):
        gmin = jnp.minimum(gmin, s[:, g * W:(g + 1) * W])

    big = jnp.float32(jnp.inf)
    t = jnp.full((R, 1), -jnp.inf, dtype=jnp.float32)
    for _ in range(K):
        m = jnp.where(gmin > t, gmin, big)
        t = jnp.min(m, axis=-1, keepdims=True)

    cnt = jnp.sum((s <= t).astype(jnp.float32), axis=-1, keepdims=True)
    over = cnt - jnp.float32(K)

    def cond(carry):
        _, over_c = carry
        return jnp.max(over_c) > 0.0

    def body(carry):
        t_c, over_c = carry
        tm = jnp.max(jnp.where(s < t_c, s, NEG), axis=-1, keepdims=True)
        need = over_c > 0.0
        return (jnp.where(need, tm, t_c),
                jnp.where(need, over_c - 1.0, over_c))

    t, _ = jax.lax.while_loop(cond, body, (t, over))

    out_ref[0] = (s <= t).astype(jnp.float32)


@jax.jit
def kernel(x):
    grid = (B, N // R)
    return pl.pallas_call(
        _adj_kernel,
        grid=grid,
        in_specs=[
            pl.BlockSpec((1, R, C), lambda b, i: (b, i, 0)),
            pl.BlockSpec((1, N, C), lambda b, i: (b, 0, 0)),
        ],
        out_specs=pl.BlockSpec((1, R, N), lambda b, i: (b, i, 0)),
        out_shape=jax.ShapeDtypeStruct((B, N, N), jnp.float32),
        compiler_params=pltpu.CompilerParams(
            dimension_semantics=("parallel", "arbitrary"),
        ),
    )(x, x)


# speedup vs baseline: 1.2185x; 1.2185x over previous
import jax
import jax.numpy as jnp
from jax.experimental import pallas as pl
from jax.experimental.pallas import tpu as pltpu

B, N, C, K = 2, 4096, 16, 16
R = 1024
G = 16
W = N // G
NEG = -1e30


def _adj_kernel(xr_ref, xa_ref, out_ref):
    xr = xr_ref[0]
    xa = xa_ref[0]
    sq_a_half = 0.5 * jnp.sum(xa * xa, axis=-1, keepdims=True).T
    dots = jax.lax.dot_general(
        xr, xa, (((1,), (1,)), ((), ())),
        preferred_element_type=jnp.float32)
    s = sq_a_half - dots

    big = jnp.float32(jnp.inf)
    a1 = s[:, :W]
    a2 = jnp.full_like(a1, big)
    for g in range(1, G):
        b = s[:, g * W:(g + 1) * W]
        hi = jnp.maximum(a1, b)
        a1 = jnp.minimum(a1, b)
        a2 = jnp.minimum(a2, hi)
    gmin = jnp.concatenate([a1, a2], axis=1)

    t = jnp.full((R, 1), -jnp.inf, dtype=jnp.float32)
    for _ in range(K):
        m = jnp.where(gmin > t, gmin, big)
        t = jnp.min(m, axis=-1, keepdims=True)

    cnt = jnp.sum((s <= t).astype(jnp.float32), axis=-1, keepdims=True)
    over = cnt - jnp.float32(K)

    def cond(carry):
        _, over_c = carry
        return jnp.max(over_c) > 0.0

    def body(carry):
        t_c, over_c = carry
        tm = jnp.max(jnp.where(s < t_c, s, NEG), axis=-1, keepdims=True)
        need = over_c > 0.0
        return (jnp.where(need, tm, t_c),
                jnp.where(need, over_c - 1.0, over_c))

    t, _ = jax.lax.while_loop(cond, body, (t, over))

    out_ref[0] = (s <= t).astype(jnp.float32)


@jax.jit
def kernel(x):
    grid = (B, N // R)
    return pl.pallas_call(
        _adj_kernel,
        grid=grid,
        in_specs=[
            pl.BlockSpec((1, R, C), lambda b, i: (b, i, 0)),
            pl.BlockSpec((1, N, C), lambda b, i: (b, 0, 0)),
        ],
        out_specs=pl.BlockSpec((1, R, N), lambda b, i: (b, i, 0)),
        out_shape=jax.ShapeDtypeStruct((B, N, N), jnp.float32),
        compiler_params=pltpu.CompilerParams(
            dimension_semantics=("parallel", "arbitrary"),
        ),
    )(x, x)


# speedup vs baseline: 1.2270x; 1.0070x over previous
import jax
import jax.numpy as jnp
from jax.experimental import pallas as pl
from jax.experimental.pallas import tpu as pltpu

B, N, C, K = 2, 4096, 16, 16
R = 1024
G = 16
W = N // G
NEG = -1e30


def _adj_kernel(xr_ref, xa_ref, out_ref):
    xr = xr_ref[0]
    xa = xa_ref[0]
    sq_a_half = 0.5 * jnp.sum(xa * xa, axis=-1, keepdims=True).T
    dots = jax.lax.dot_general(
        xr, xa, (((1,), (1,)), ((), ())),
        preferred_element_type=jnp.float32)
    s = sq_a_half - dots

    big = jnp.float32(jnp.inf)
    a1 = s[:, :W]
    a2 = jnp.full_like(a1, big)
    for g in range(1, G):
        b = s[:, g * W:(g + 1) * W]
        hi = jnp.maximum(a1, b)
        a1 = jnp.minimum(a1, b)
        a2 = jnp.minimum(a2, hi)
    gmin = jnp.concatenate([a1, a2], axis=1)

    t = jnp.full((R, 1), -jnp.inf, dtype=jnp.float32)
    for _ in range(K):
        m = jnp.where(gmin > t, gmin, big)
        t = jnp.min(m, axis=-1, keepdims=True)

    cnt = jnp.sum((s <= t).astype(jnp.float32), axis=-1, keepdims=True)
    over = cnt - jnp.float32(K)

    SUB = 8
    RS = R // SUB
    t_parts = []
    for u in range(SUB):
        s_u = s[u * RS:(u + 1) * RS]
        t_u = t[u * RS:(u + 1) * RS]
        over_u = over[u * RS:(u + 1) * RS]

        def cond(carry):
            _, over_c = carry
            return jnp.max(over_c) > 0.0

        def body(carry, s_ref=s_u):
            t_c, over_c = carry
            tm = jnp.max(jnp.where(s_ref < t_c, s_ref, NEG), axis=-1,
                         keepdims=True)
            need = over_c > 0.0
            return (jnp.where(need, tm, t_c),
                    jnp.where(need, over_c - 1.0, over_c))

        t_u, _ = jax.lax.while_loop(cond, body, (t_u, over_u))
        t_parts.append(t_u)
    t = jnp.concatenate(t_parts, axis=0)

    out_ref[0] = (s <= t).astype(jnp.float32)


@jax.jit
def kernel(x):
    grid = (B, N // R)
    return pl.pallas_call(
        _adj_kernel,
        grid=grid,
        in_specs=[
            pl.BlockSpec((1, R, C), lambda b, i: (b, i, 0)),
            pl.BlockSpec((1, N, C), lambda b, i: (b, 0, 0)),
        ],
        out_specs=pl.BlockSpec((1, R, N), lambda b, i: (b, i, 0)),
        out_shape=jax.ShapeDtypeStruct((B, N, N), jnp.float32),
        compiler_params=pltpu.CompilerParams(
            dimension_semantics=("parallel", "arbitrary"),
        ),
    )(x, x)


# speedup vs baseline: 1.3200x; 1.0758x over previous
import jax
import jax.numpy as jnp
from jax.experimental import pallas as pl
from jax.experimental.pallas import tpu as pltpu

B, N, C, K = 2, 4096, 16, 16
R = 1024
G = 16
W = N // G
POS = 1e30


def _adj_kernel(xr_ref, xa_ref, out_ref):
    xr = xr_ref[0]
    xa = xa_ref[0]
    p = jax.lax.dot_general(
        xr, xa, (((1,), (1,)), ((), ())),
        preferred_element_type=jnp.float32)

    big = jnp.float32(-jnp.inf)
    a1 = p[:, :W]
    a2 = jnp.full_like(a1, big)
    for g in range(1, G):
        b = p[:, g * W:(g + 1) * W]
        lo = jnp.minimum(a1, b)
        a1 = jnp.maximum(a1, b)
        a2 = jnp.maximum(a2, lo)
    gmax = jnp.concatenate([a1, a2], axis=1)

    t = jnp.full((R, 1), jnp.inf, dtype=jnp.float32)
    for _ in range(K):
        m = jnp.where(gmax < t, gmax, big)
        t = jnp.max(m, axis=-1, keepdims=True)

    mask0 = (p >= t).astype(jnp.float32)
    out_ref[0] = mask0
    cnt = jnp.sum(mask0, axis=-1, keepdims=True)
    over = cnt - jnp.float32(K)

    SUB = 8
    RS = R // SUB
    for u in range(SUB):
        p_u = p[u * RS:(u + 1) * RS]
        t_u = t[u * RS:(u + 1) * RS]
        over_u = over[u * RS:(u + 1) * RS]

        def cond(carry):
            _, over_c = carry
            return jnp.max(over_c) > 0.0

        def body(carry, p_ref=p_u):
            t_c, over_c = carry
            tm = jnp.min(jnp.where(p_ref > t_c, p_ref, POS), axis=-1,
                         keepdims=True)
            need = over_c > 0.0
            return (jnp.where(need, tm, t_c),
                    jnp.where(need, over_c - 1.0, over_c))

        t_fix, _ = jax.lax.while_loop(cond, body, (t_u, over_u))

        @pl.when(jnp.max(over_u) > 0.0)
        def _fix(p_ref=p_u, t_f=t_fix, u=u):
            out_ref[0, u * RS:(u + 1) * RS, :] = (
                p_ref >= t_f).astype(jnp.float32)


@jax.jit
def kernel(x):
    sqh = -0.5 * jnp.sum(x * x, axis=-1, keepdims=True)
    xa_aug = jnp.concatenate([x, sqh], axis=-1)
    xr_aug = jnp.concatenate(
        [x, jnp.ones((B, N, 1), dtype=x.dtype)], axis=-1)
    grid = (B, N // R)
    return pl.pallas_call(
        _adj_kernel,
        grid=grid,
        in_specs=[
            pl.BlockSpec((1, R, C + 1), lambda b, i: (b, i, 0)),
            pl.BlockSpec((1, N, C + 1), lambda b, i: (b, 0, 0)),
        ],
        out_specs=pl.BlockSpec((1, R, N), lambda b, i: (b, i, 0)),
        out_shape=jax.ShapeDtypeStruct((B, N, N), jnp.float32),
        compiler_params=pltpu.CompilerParams(
            dimension_semantics=("parallel", "arbitrary"),
        ),
    )(xr_aug, xa_aug)
